# baseline (speedup 1.0000x reference)
import jax
import jax.numpy as jnp
from jax import lax
from jax.experimental import pallas as pl
from jax.experimental.pallas import tpu as pltpu
from jax.experimental.pallas import tpu_sc as plsc
import functools

N_FACTORS = 64
BATCH = 16384
NC = 2
NS = 16
LANES = 16
NW = NC * NS
B_PER_W = BATCH // NW
CHUNK = 128
N_CHUNKS = B_PER_W // CHUNK
N_GROUPS = B_PER_W // LANES


def _mf_kernel(uid_hbm, iid_hbm, uvec_hbm, ivec_hbm, ubias_hbm, ibias_hbm,
               out_hbm, uid_v, iid_v, u_rows, i_rows, ub_v, ib_v, out_v, sem):
    wid = lax.axis_index("s") * NC + lax.axis_index("c")
    base = wid * B_PER_W

    pltpu.sync_copy(uid_hbm.at[wid], uid_v)
    pltpu.sync_copy(iid_hbm.at[wid], iid_v)

    copies = []
    for j in range(N_CHUNKS):
        dst = pl.ds(j * CHUNK, CHUNK)
        copies.append(pltpu.async_copy(uvec_hbm.at[uid_v.at[j]], u_rows.at[dst], sem))
        copies.append(pltpu.async_copy(ivec_hbm.at[iid_v.at[j]], i_rows.at[dst], sem))
        copies.append(pltpu.async_copy(ubias_hbm.at[uid_v.at[j]], ub_v.at[dst], sem))
        copies.append(pltpu.async_copy(ibias_hbm.at[iid_v.at[j]], ib_v.at[dst], sem))
    for c in copies:
        c.wait()

    lane = lax.iota(jnp.int32, LANES)

    def group_body(g, _):
        row_idx = g * LANES + lane
        accs = [ub_v[pl.ds(g * LANES, LANES)] + ib_v[pl.ds(g * LANES, LANES)],
                jnp.zeros((LANES,), jnp.float32),
                jnp.zeros((LANES,), jnp.float32),
                jnp.zeros((LANES,), jnp.float32)]
        for f in range(N_FACTORS):
            col = jnp.full((LANES,), f, jnp.int32)
            ug = plsc.load_gather(u_rows, [row_idx, col])
            ig = plsc.load_gather(i_rows, [row_idx, col])
            accs[f % 4] = accs[f % 4] + ug * ig
        out_v[pl.ds(g * LANES, LANES)] = (accs[0] + accs[1]) + (accs[2] + accs[3])
        return _

    lax.fori_loop(0, N_GROUPS, group_body, None)

    pltpu.sync_copy(out_v, out_hbm.at[pl.ds(base, B_PER_W)])


@jax.jit
def _mf(uid2, iid2, users_vectors, items_vectors, ub1, ib1):
    mesh = plsc.VectorSubcoreMesh(core_axis_name="c", subcore_axis_name="s",
                                  num_cores=NC, num_subcores=NS)
    return pl.kernel(
        _mf_kernel,
        out_type=jax.ShapeDtypeStruct((BATCH,), jnp.float32),
        mesh=mesh,
        compiler_params=pltpu.CompilerParams(
            needs_layout_passes=False, use_tc_tiling_on_sc=False),
        scratch_types=[
            pltpu.VMEM((N_CHUNKS, CHUNK), jnp.int32),
            pltpu.VMEM((N_CHUNKS, CHUNK), jnp.int32),
            pltpu.VMEM((B_PER_W, N_FACTORS), jnp.float32),
            pltpu.VMEM((B_PER_W, N_FACTORS), jnp.float32),
            pltpu.VMEM((B_PER_W,), jnp.float32),
            pltpu.VMEM((B_PER_W,), jnp.float32),
            pltpu.VMEM((B_PER_W,), jnp.float32),
            pltpu.SemaphoreType.DMA,
        ],
    )(uid2, iid2, users_vectors, items_vectors, ub1, ib1)


def kernel(user_id, item_id, users_vectors, items_vectors, users_bias, items_bias):
    uid2 = user_id.reshape(NW, N_CHUNKS, CHUNK)
    iid2 = item_id.reshape(NW, N_CHUNKS, CHUNK)
    ub1 = users_bias.reshape(-1)
    ib1 = items_bias.reshape(-1)
    return _mf(uid2, iid2, users_vectors, items_vectors, ub1, ib1)
